# baseline (speedup 1.0000x reference)
import functools

import jax
import jax.numpy as jnp
from jax import lax
from jax.experimental import pallas as pl
from jax.experimental.pallas import tpu as pltpu
from jax.experimental.pallas import tpu_sc as plsc

_BATCH, _SEQ, _DIM = 4096, 50, 64
_N = _BATCH * _SEQ
_NW = 32
_TPW = _N // _NW
_C = 160
_NCHUNK = _TPW // _C
_G = _C // 16


def _sc_embed(ids, pool2, hv0_tab, hv1_tab, ip0_tab, ip1_tab):
    mesh = plsc.VectorSubcoreMesh(core_axis_name="c", subcore_axis_name="s")

    @functools.partial(
        pl.kernel,
        mesh=mesh,
        compiler_params=pltpu.CompilerParams(use_tc_tiling_on_sc=False),
        out_type=jax.ShapeDtypeStruct((_N, _DIM), jnp.float32),
        scratch_types=[
            pltpu.VMEM((_TPW,), jnp.int32),
            [pltpu.VMEM((2 * _C,), jnp.int32) for _ in range(4)],
            [pltpu.VMEM((2 * _C,), jnp.int32) for _ in range(4)],
            [pltpu.VMEM((_C,), jnp.float32) for _ in range(4)],
            [pltpu.VMEM((_C,), jnp.float32) for _ in range(4)],
            [pltpu.VMEM((2 * _C, 128), jnp.float32) for _ in range(2)],
            [pltpu.VMEM((_C, _DIM), jnp.float32) for _ in range(2)],
            [pltpu.SemaphoreType.DMA for _ in range(4)],
            [pltpu.SemaphoreType.DMA for _ in range(2)],
            [pltpu.SemaphoreType.DMA for _ in range(2)],
        ],
    )
    def k(ids_hbm, pool_hbm, hv0_hbm, hv1_hbm, ip0_hbm, ip1_hbm, out_hbm,
          ids_v, hv_v, sel_v, ip0_v, ip1_v, r_v, o_v,
          sem_side, sem_rows, sem_out):
        wid = lax.axis_index("s") * 2 + lax.axis_index("c")
        t0w = wid * _TPW

        pltpu.sync_copy(ids_hbm.at[pl.ds(t0w, _TPW)], ids_v)


        def side_cps(n, p, make=False):
            f = pltpu.make_async_copy if make else pltpu.async_copy
            idx = ids_v.at[pl.ds(n * _C, _C)]
            return [
                f(hv0_hbm.at[idx], hv_v[p].at[pl.ds(0, _C)], sem_side[p]),
                f(hv1_hbm.at[idx], hv_v[p].at[pl.ds(_C, _C)], sem_side[p]),
                f(ip0_hbm.at[idx], ip0_v[p], sem_side[p]),
                f(ip1_hbm.at[idx], ip1_v[p], sem_side[p]),
            ]

        def rows_cp(p4, pr, make=False):
            f = pltpu.make_async_copy if make else pltpu.async_copy
            return f(pool_hbm.at[hv_v[p4]], r_v[pr], sem_rows[pr])

        def out_cp(n, pr, make=False):
            f = pltpu.make_async_copy if make else pltpu.async_copy
            return f(o_v[pr], out_hbm.at[pl.ds(t0w + n * _C, _C)],
                     sem_out[pr])

        def xform(p):
            for g in range(2 * _G):
                sl = pl.ds(g * 16, 16)
                h = hv_v[p][sl]
                hv_v[p][sl] = lax.shift_right_logical(h, 1)
                sel_v[p][sl] = (h & 1) * 64

        def compute(p4, pr):
            rp, op = r_v[pr], o_v[pr]
            ip0p, ip1p, selp = ip0_v[p4], ip1_v[p4], sel_v[p4]

            def group_body(g, carry):
                t0 = g * 16
                wv0 = ip0p[pl.ds(t0, 16)]
                wv1 = ip1p[pl.ds(t0, 16)]
                sv0 = selp[pl.ds(t0, 16)]
                sv1 = selp[pl.ds(_C + t0, 16)]
                for j in range(16):
                    t = t0 + j
                    w0 = wv0[j]
                    w1 = wv1[j]
                    s0 = sv0[j]
                    s1 = sv1[j]
                    for q in range(4):
                        a = rp[t, pl.ds(s0 + q * 16, 16)]
                        b = rp[_C + t, pl.ds(s1 + q * 16, 16)]
                        op[t, pl.ds(q * 16, 16)] = w0 * a + w1 * b
                return carry

            lax.fori_loop(0, _G, group_body, 0)

        def step(n, c):
            if c + 2 < _NCHUNK:
                side_cps(n + 2, (c + 2) % 4)
            if c + 1 < _NCHUNK:
                for cp in side_cps(n + 1, (c + 1) % 4, make=True):
                    cp.wait()
                xform((c + 1) % 4)
                if c >= 1:
                    out_cp(n - 1, (c - 1) % 2, make=True).wait()
                rows_cp((c + 1) % 4, (c + 1) % 2)
            rows_cp(c % 4, c % 2, make=True).wait()
            compute(c % 4, c % 2)
            out_cp(n, c % 2)

        side_cps(0, 0)
        side_cps(1, 1)
        for cp in side_cps(0, 0, make=True):
            cp.wait()
        xform(0)
        rows_cp(0, 0)
        step(0, 0)

        def steady(i, carry):
            base = jnp.int32(1) + i * 4
            for u in range(4):
                step(base + u, 1 + u)
            return carry

        lax.fori_loop(0, 9, steady, 0)

        for c in (37, 38, 39):
            step(c, c)

        out_cp(38, 0, make=True).wait()
        out_cp(39, 1, make=True).wait()

    return k(ids, pool2, hv0_tab, hv1_tab, ip0_tab, ip1_tab)


def kernel(input, pool_weight, import_params, hash_values):
    ids = input.reshape(_N).astype(jnp.int32)
    pool2 = pool_weight.reshape(100000 * _DIM // 128, 128)
    hv0_tab = hash_values[:, 0].astype(jnp.int32)
    hv1_tab = hash_values[:, 1].astype(jnp.int32)
    ip0_tab = import_params[:, 0] * 1.0
    ip1_tab = import_params[:, 1] * 1.0
    out = _sc_embed(ids, pool2, hv0_tab, hv1_tab, ip0_tab, ip1_tab)
    return out.reshape(_BATCH, _SEQ, _DIM)


# speedup vs baseline: 1.0639x; 1.0639x over previous
import functools

import jax
import jax.numpy as jnp
from jax import lax
from jax.experimental import pallas as pl
from jax.experimental.pallas import tpu as pltpu
from jax.experimental.pallas import tpu_sc as plsc

_BATCH, _SEQ, _DIM = 4096, 50, 64
_N = _BATCH * _SEQ
_V = 100000
_NW = 32
_TPW = _N // _NW
_C = 100
_NCHUNK = _TPW // _C
_GF = _C // 16
_CT = _C - 16 * _GF


def _sc_embed(ids, pool2, hv0_tab, hv1_tab, ip0_tab, ip1_tab):
    mesh = plsc.VectorSubcoreMesh(core_axis_name="c", subcore_axis_name="s")

    @functools.partial(
        pl.kernel,
        mesh=mesh,
        out_type=jax.ShapeDtypeStruct((_BATCH, _SEQ, _DIM), jnp.float32),
        scratch_types=[
            pltpu.VMEM((_TPW // _C, _C), jnp.int32),
            [pltpu.VMEM((224,), jnp.int32) for _ in range(4)],
            [pltpu.VMEM((224,), jnp.int32) for _ in range(4)],
            [pltpu.VMEM((112,), jnp.float32) for _ in range(4)],
            [pltpu.VMEM((112,), jnp.float32) for _ in range(4)],
            [pltpu.VMEM((208, 128), jnp.float32) for _ in range(2)],
            [pltpu.VMEM((_C, _DIM), jnp.float32) for _ in range(2)],
            pltpu.VMEM_SHARED((_V,), jnp.int32),
            pltpu.VMEM_SHARED((_V,), jnp.int32),
            pltpu.VMEM_SHARED((_V,), jnp.float32),
            pltpu.VMEM_SHARED((_V,), jnp.float32),
            pltpu.SemaphoreType.DMA,
            [pltpu.SemaphoreType.DMA for _ in range(4)],
            [pltpu.SemaphoreType.DMA for _ in range(2)],
            [pltpu.SemaphoreType.DMA for _ in range(2)],
            pltpu.VMEM((6400,), jnp.int32),
            pltpu.VMEM((6400,), jnp.float32),
        ],
    )
    def k(ids_hbm, pool_hbm, hv0_hbm, hv1_hbm, ip0_hbm, ip1_hbm, out_hbm,
          ids_v, hv_v, sel_v, ip0_v, ip1_v, r_v, o_v,
          hv0_sp, hv1_sp, ip0_sp, ip1_sp,
          sem_st, sem_side, sem_rows, sem_out, bounce_v, bounce_f):
        cid = lax.axis_index("c")
        sid = lax.axis_index("s")
        wid = sid * 2 + cid
        t0w = wid * _TPW
        b0w = wid * (_TPW // _SEQ)

        pltpu.sync_copy(ids_hbm.at[pl.ds(wid * (_TPW // _C), _TPW // _C)],
                        ids_v)

        @pl.when(sid < 15)
        def _():
            off = sid * 6400
            for tab, sp in ((hv0_hbm, hv0_sp), (hv1_hbm, hv1_sp)):
                pltpu.sync_copy(tab.at[pl.ds(off, 6400)], bounce_v)
                pltpu.sync_copy(bounce_v, sp.at[pl.ds(off, 6400)])
            for tab, sp in ((ip0_hbm, ip0_sp), (ip1_hbm, ip1_sp)):
                pltpu.sync_copy(tab.at[pl.ds(off, 6400)], bounce_f)
                pltpu.sync_copy(bounce_f, sp.at[pl.ds(off, 6400)])

        @pl.when(sid == 15)
        def _():
            for tab, sp in ((hv0_hbm, hv0_sp), (hv1_hbm, hv1_sp)):
                pltpu.sync_copy(tab.at[pl.ds(96000, 4000)],
                                bounce_v.at[pl.ds(0, 4000)])
                pltpu.sync_copy(bounce_v.at[pl.ds(0, 4000)],
                                sp.at[pl.ds(96000, 4000)])
            for tab, sp in ((ip0_hbm, ip0_sp), (ip1_hbm, ip1_sp)):
                pltpu.sync_copy(tab.at[pl.ds(96000, 4000)],
                                bounce_f.at[pl.ds(0, 4000)])
                pltpu.sync_copy(bounce_f.at[pl.ds(0, 4000)],
                                sp.at[pl.ds(96000, 4000)])

        plsc.subcore_barrier()


        def side_cps(n, p, make=False):
            f = pltpu.make_async_copy if make else pltpu.async_copy
            idx = ids_v.at[n]
            return [
                f(hv0_sp.at[idx], hv_v[p].at[pl.ds(0, _C)], sem_side[p]),
                f(hv1_sp.at[idx], hv_v[p].at[pl.ds(104, _C)], sem_side[p]),
                f(ip0_sp.at[idx], ip0_v[p].at[pl.ds(0, _C)], sem_side[p]),
                f(ip1_sp.at[idx], ip1_v[p].at[pl.ds(0, _C)], sem_side[p]),
            ]

        def rows_cp(p4, pr, make=False):
            f = pltpu.make_async_copy if make else pltpu.async_copy
            return [
                f(pool_hbm.at[hv_v[p4].at[pl.ds(0, _C)]],
                  r_v[pr].at[pl.ds(0, _C)], sem_rows[pr]),
                f(pool_hbm.at[hv_v[p4].at[pl.ds(104, _C)]],
                  r_v[pr].at[pl.ds(104, _C)], sem_rows[pr]),
            ]

        def out_cps(n, pr, make=False):
            f = pltpu.make_async_copy if make else pltpu.async_copy
            b = b0w + n * 2
            return [
                f(o_v[pr].at[pl.ds(0, _SEQ)], out_hbm.at[b], sem_out[pr]),
                f(o_v[pr].at[pl.ds(_SEQ, _SEQ)], out_hbm.at[b + 1],
                  sem_out[pr]),
            ]

        def xform(p):
            for g in range(208 // 16):
                sl = pl.ds(g * 16, 16)
                h = hv_v[p][sl]
                hv_v[p][sl] = lax.shift_right_logical(h, 1)
                sel_v[p][sl] = (h & 1) * 64

        def combine(rp, op, t, w0, w1, s0, s1):
            for q in range(4):
                a = rp[t, pl.ds(s0 + q * 16, 16)]
                b = rp[104 + t, pl.ds(s1 + q * 16, 16)]
                op[t, pl.ds(q * 16, 16)] = w0 * a + w1 * b

        def compute(p4, pr):
            rp, op = r_v[pr], o_v[pr]
            ip0p, ip1p, selp = ip0_v[p4], ip1_v[p4], sel_v[p4]

            def group_body(g, carry):
                t0 = g * 16
                wv0 = ip0p[pl.ds(t0, 16)]
                wv1 = ip1p[pl.ds(t0, 16)]
                sv0 = selp[pl.ds(t0, 16)]
                sv1 = selp[pl.ds(104 + t0, 16)]
                for j in range(16):
                    combine(rp, op, t0 + j, wv0[j], wv1[j], sv0[j], sv1[j])
                return carry

            lax.fori_loop(0, _GF, group_body, 0)
            t0 = 16 * _GF
            wv0 = ip0p[pl.ds(t0, 16)]
            wv1 = ip1p[pl.ds(t0, 16)]
            sv0 = selp[pl.ds(t0, 16)]
            sv1 = selp[pl.ds(104 + t0, 16)]
            for j in range(_CT):
                combine(rp, op, t0 + j, wv0[j], wv1[j], sv0[j], sv1[j])

        def step(n, c):
            if c + 2 < _NCHUNK:
                side_cps(n + 2, (c + 2) % 4)
            if c + 1 < _NCHUNK:
                for cp in side_cps(n + 1, (c + 1) % 4, make=True):
                    cp.wait()
                xform((c + 1) % 4)
                if c >= 1:
                    for cp in out_cps(n - 1, (c - 1) % 2, make=True):
                        cp.wait()
                rows_cp((c + 1) % 4, (c + 1) % 2)
            for cp in rows_cp(c % 4, c % 2, make=True):
                cp.wait()
            compute(c % 4, c % 2)
            out_cps(n, c % 2)

        side_cps(0, 0)
        side_cps(1, 1)
        for cp in side_cps(0, 0, make=True):
            cp.wait()
        xform(0)
        rows_cp(0, 0)
        step(0, 0)

        def steady(i, carry):
            base = jnp.int32(1) + i * 4
            for u in range(4):
                step(base + u, 1 + u)
            return carry

        lax.fori_loop(0, 15, steady, 0)

        for c in (61, 62, 63):
            step(c, c)

        for cp in out_cps(62, 0, make=True):
            cp.wait()
        for cp in out_cps(63, 1, make=True):
            cp.wait()

    return k(ids, pool2, hv0_tab, hv1_tab, ip0_tab, ip1_tab)


def kernel(input, pool_weight, import_params, hash_values):
    ids = input.reshape(_N // _C, _C).astype(jnp.int32)
    pool2 = pool_weight.reshape(_V * _DIM // 128, 128)
    hv0_tab = hash_values[:, 0].astype(jnp.int32)
    hv1_tab = hash_values[:, 1].astype(jnp.int32)
    ip0_tab = import_params[:, 0] * 1.0
    ip1_tab = import_params[:, 1] * 1.0
    return _sc_embed(ids, pool2, hv0_tab, hv1_tab, ip0_tab, ip1_tab)
